# Initial kernel scaffold; baseline (speedup 1.0000x reference)
import jax
import jax.numpy as jnp
from jax.experimental import pallas as pl


def kernel(X, W, b, Bmat):
    raise NotImplementedError("write your pallas kernel here")


# speedup vs baseline: 7.1636x; 7.1636x over previous
import jax
import jax.numpy as jnp
from jax.experimental import pallas as pl
from jax.experimental.pallas import tpu as pltpu

_B, _N, _DIN, _RANK, _TOPK = 4, 2048, 256, 64, 32
_BLK = 256
_PREC = jax.lax.Precision.DEFAULT


def _r_body(x_ref, w_ref, b_ref, r_ref):
    x = x_ref[0]
    r = jax.lax.dot_general(x, w_ref[...], (((1,), (0,)), ((), ())),
                            preferred_element_type=jnp.float32,
                            precision=_PREC)
    r_ref[0] = r + b_ref[...]


def _s_body(rblk_ref, rfull_ref, bmat_ref, s_ref, a_ref):
    rblk = rblk_ref[0]
    rfull = rfull_ref[0]
    p = jax.lax.dot_general(rblk, bmat_ref[...], (((1,), (0,)), ((), ())),
                            preferred_element_type=jnp.float32,
                            precision=_PREC)
    s = jax.lax.dot_general(p, rfull, (((1,), (1,)), ((), ())),
                            preferred_element_type=jnp.float32,
                            precision=_PREC)
    s_ref[0] = s

    m = jnp.max(s, axis=-1, keepdims=True)
    e = jnp.exp(s - m)
    z = jnp.sum(e, axis=-1, keepdims=True)

    rows = pl.program_id(1) * _BLK + jax.lax.broadcasted_iota(
        jnp.int32, (_BLK, _N), 0)
    cols = jax.lax.broadcasted_iota(jnp.int32, (_BLK, _N), 1)
    nondiag = rows != cols
    neginf = jnp.float32(-jnp.inf)
    work = jnp.where(nondiag, s, neginf)

    def body(_, w):
        v = jnp.max(w, axis=-1, keepdims=True)
        return jnp.where(w >= v, neginf, w)

    work = jax.lax.fori_loop(0, _TOPK - 1, body, work)
    t = jnp.max(work, axis=-1, keepdims=True)

    keep = nondiag & (s >= t)
    ek = jnp.where(keep, e, jnp.float32(0.0))
    s1 = jnp.sum(ek, axis=-1, keepdims=True) / z
    c1 = jnp.maximum(s1, jnp.float32(1e-6))
    c2 = jnp.maximum(s1 / c1, jnp.float32(1e-6))
    scale = 1.0 / (z * c1 * c2)
    a_ref[0] = jnp.where(keep, e * scale, jnp.float32(0.0))


def kernel(X, W, b, Bmat):
    R = pl.pallas_call(
        _r_body,
        grid=(_B,),
        in_specs=[
            pl.BlockSpec((1, _N, _DIN), lambda i: (i, 0, 0)),
            pl.BlockSpec((_DIN, _RANK), lambda i: (0, 0)),
            pl.BlockSpec((1, _RANK), lambda i: (0, 0)),
        ],
        out_specs=pl.BlockSpec((1, _N, _RANK), lambda i: (i, 0, 0)),
        out_shape=jax.ShapeDtypeStruct((_B, _N, _RANK), jnp.float32),
    )(X, W, b.reshape(1, _RANK))

    S, A = pl.pallas_call(
        _s_body,
        grid=(_B, _N // _BLK),
        in_specs=[
            pl.BlockSpec((1, _BLK, _RANK), lambda i, j: (i, j, 0)),
            pl.BlockSpec((1, _N, _RANK), lambda i, j: (i, 0, 0)),
            pl.BlockSpec((_RANK, _RANK), lambda i, j: (0, 0)),
        ],
        out_specs=[
            pl.BlockSpec((1, _BLK, _N), lambda i, j: (i, j, 0)),
            pl.BlockSpec((1, _BLK, _N), lambda i, j: (i, j, 0)),
        ],
        out_shape=[
            jax.ShapeDtypeStruct((_B, _N, _N), jnp.float32),
            jax.ShapeDtypeStruct((_B, _N, _N), jnp.float32),
        ],
    )(R, R, Bmat)
    return (A, S, R)


# speedup vs baseline: 16.7410x; 2.3369x over previous
import jax
import jax.numpy as jnp
from jax.experimental import pallas as pl
from jax.experimental.pallas import tpu as pltpu

_B, _N, _DIN, _RANK, _TOPK = 4, 2048, 256, 64, 32
_BLK = 256
_PREC = jax.lax.Precision.DEFAULT


def _r_body(x_ref, w_ref, b_ref, r_ref):
    x = x_ref[0]
    r = jax.lax.dot_general(x, w_ref[...], (((1,), (0,)), ((), ())),
                            preferred_element_type=jnp.float32,
                            precision=_PREC)
    r_ref[0] = r + b_ref[...]


def _s_body(rblk_ref, rfull_ref, bmat_ref, s_ref, a_ref):
    rblk = rblk_ref[0]
    rfull = rfull_ref[0]
    p = jax.lax.dot_general(rblk, bmat_ref[...], (((1,), (0,)), ((), ())),
                            preferred_element_type=jnp.float32,
                            precision=_PREC)
    s = jax.lax.dot_general(p, rfull, (((1,), (1,)), ((), ())),
                            preferred_element_type=jnp.float32,
                            precision=_PREC)
    s_ref[0] = s

    m = jnp.max(s, axis=-1, keepdims=True)
    e = jnp.exp(s - m)
    z = jnp.sum(e, axis=-1, keepdims=True)

    rows = pl.program_id(1) * _BLK + jax.lax.broadcasted_iota(
        jnp.int32, (_BLK, _N), 0)
    cols = jax.lax.broadcasted_iota(jnp.int32, (_BLK, _N), 1)
    nondiag = rows != cols
    neginf = jnp.float32(-jnp.inf)
    work = jnp.where(nondiag, s, neginf)

    def body(_, w):
        v = jnp.max(w, axis=-1, keepdims=True)
        return jnp.where(w >= v, neginf, w)

    work = jax.lax.fori_loop(0, _TOPK - 1, body, work, unroll=True)
    t = jnp.max(work, axis=-1, keepdims=True)

    keep = nondiag & (s >= t)
    ek = jnp.where(keep, e, jnp.float32(0.0))
    s1 = jnp.sum(ek, axis=-1, keepdims=True) / z
    c1 = jnp.maximum(s1, jnp.float32(1e-6))
    c2 = jnp.maximum(s1 / c1, jnp.float32(1e-6))
    scale = 1.0 / (z * c1 * c2)
    a_ref[0] = jnp.where(keep, e * scale, jnp.float32(0.0))


def kernel(X, W, b, Bmat):
    R = pl.pallas_call(
        _r_body,
        grid=(_B,),
        in_specs=[
            pl.BlockSpec((1, _N, _DIN), lambda i: (i, 0, 0)),
            pl.BlockSpec((_DIN, _RANK), lambda i: (0, 0)),
            pl.BlockSpec((1, _RANK), lambda i: (0, 0)),
        ],
        out_specs=pl.BlockSpec((1, _N, _RANK), lambda i: (i, 0, 0)),
        out_shape=jax.ShapeDtypeStruct((_B, _N, _RANK), jnp.float32),
    )(X, W, b.reshape(1, _RANK))

    S, A = pl.pallas_call(
        _s_body,
        grid=(_B, _N // _BLK),
        in_specs=[
            pl.BlockSpec((1, _BLK, _RANK), lambda i, j: (i, j, 0)),
            pl.BlockSpec((1, _N, _RANK), lambda i, j: (i, 0, 0)),
            pl.BlockSpec((_RANK, _RANK), lambda i, j: (0, 0)),
        ],
        out_specs=[
            pl.BlockSpec((1, _BLK, _N), lambda i, j: (i, j, 0)),
            pl.BlockSpec((1, _BLK, _N), lambda i, j: (i, j, 0)),
        ],
        out_shape=[
            jax.ShapeDtypeStruct((_B, _N, _N), jnp.float32),
            jax.ShapeDtypeStruct((_B, _N, _N), jnp.float32),
        ],
    )(R, R, Bmat)
    return (A, S, R)


# speedup vs baseline: 23.6171x; 1.4107x over previous
import jax
import jax.numpy as jnp
from jax.experimental import pallas as pl
from jax.experimental.pallas import tpu as pltpu

_B, _N, _DIN, _RANK, _TOPK = 4, 2048, 256, 64, 32
_BLK = 256
_PREC = jax.lax.Precision.DEFAULT


def _r_body(x_ref, w_ref, b_ref, r_ref):
    x = x_ref[0]
    r = jax.lax.dot_general(x, w_ref[...], (((1,), (0,)), ((), ())),
                            preferred_element_type=jnp.float32,
                            precision=_PREC)
    r_ref[0] = r + b_ref[...]


def _s_body(rblk_ref, rfull_ref, bmat_ref, s_ref, a_ref):
    rblk = rblk_ref[0]
    rfull = rfull_ref[0]
    p = jax.lax.dot_general(rblk, bmat_ref[...], (((1,), (0,)), ((), ())),
                            preferred_element_type=jnp.float32,
                            precision=_PREC)
    s = jax.lax.dot_general(p, rfull, (((1,), (1,)), ((), ())),
                            preferred_element_type=jnp.float32,
                            precision=_PREC)
    s_ref[0] = s

    m = jnp.max(s, axis=-1, keepdims=True)
    e = jnp.exp(s - m)
    z = jnp.sum(e, axis=-1, keepdims=True)

    rows = pl.program_id(1) * _BLK + jax.lax.broadcasted_iota(
        jnp.int32, (_BLK, _N), 0)
    cols = jax.lax.broadcasted_iota(jnp.int32, (_BLK, _N), 1)
    nondiag = rows != cols
    neginf = jnp.float32(-jnp.inf)
    work = jnp.where(nondiag, s, neginf)

    sl = [work[:, j * 128:(j + 1) * 128] for j in range(16)]

    def _sort8_desc(v):
        net = [(0, 1), (2, 3), (4, 5), (6, 7),
               (0, 2), (1, 3), (4, 6), (5, 7),
               (1, 2), (5, 6), (0, 4), (3, 7),
               (1, 5), (2, 6), (1, 4), (3, 6),
               (2, 4), (3, 5), (3, 4)]
        v = list(v)
        for i, j in net:
            hi = jnp.maximum(v[i], v[j])
            lo = jnp.minimum(v[i], v[j])
            v[i], v[j] = hi, lo
        return v

    sa = _sort8_desc(sl[:8])
    sb = _sort8_desc(sl[8:])
    cand = [jnp.maximum(sa[i], sb[7 - i]) for i in range(8)]
    tt = jnp.concatenate(cand, axis=-1)

    for _ in range(_TOPK - 1):
        v = jnp.max(tt, axis=-1, keepdims=True)
        tt = jnp.where(tt >= v, neginf, tt)
    t_fast = jnp.max(tt, axis=-1, keepdims=True)

    cnt = jnp.sum(jnp.where(nondiag & (s >= t_fast), 1, 0).astype(jnp.int32),
                  axis=-1, keepdims=True)
    ok = jnp.all(cnt == _TOPK)

    def _slow(w):
        def body(_, ww):
            vv = jnp.max(ww, axis=-1, keepdims=True)
            return jnp.where(ww >= vv, neginf, ww)
        ww = jax.lax.fori_loop(0, _TOPK - 1, body, w)
        return jnp.max(ww, axis=-1, keepdims=True)

    t = jax.lax.cond(ok, lambda w: t_fast, _slow, work)

    keep = nondiag & (s >= t)
    ek = jnp.where(keep, e, jnp.float32(0.0))
    s1 = jnp.sum(ek, axis=-1, keepdims=True) / z
    c1 = jnp.maximum(s1, jnp.float32(1e-6))
    c2 = jnp.maximum(s1 / c1, jnp.float32(1e-6))
    scale = 1.0 / (z * c1 * c2)
    a_ref[0] = jnp.where(keep, e * scale, jnp.float32(0.0))


def kernel(X, W, b, Bmat):
    R = pl.pallas_call(
        _r_body,
        grid=(_B,),
        in_specs=[
            pl.BlockSpec((1, _N, _DIN), lambda i: (i, 0, 0)),
            pl.BlockSpec((_DIN, _RANK), lambda i: (0, 0)),
            pl.BlockSpec((1, _RANK), lambda i: (0, 0)),
        ],
        out_specs=pl.BlockSpec((1, _N, _RANK), lambda i: (i, 0, 0)),
        out_shape=jax.ShapeDtypeStruct((_B, _N, _RANK), jnp.float32),
    )(X, W, b.reshape(1, _RANK))

    S, A = pl.pallas_call(
        _s_body,
        grid=(_B, _N // _BLK),
        in_specs=[
            pl.BlockSpec((1, _BLK, _RANK), lambda i, j: (i, j, 0)),
            pl.BlockSpec((1, _N, _RANK), lambda i, j: (i, 0, 0)),
            pl.BlockSpec((_RANK, _RANK), lambda i, j: (0, 0)),
        ],
        out_specs=[
            pl.BlockSpec((1, _BLK, _N), lambda i, j: (i, j, 0)),
            pl.BlockSpec((1, _BLK, _N), lambda i, j: (i, j, 0)),
        ],
        out_shape=[
            jax.ShapeDtypeStruct((_B, _N, _N), jnp.float32),
            jax.ShapeDtypeStruct((_B, _N, _N), jnp.float32),
        ],
    )(R, R, Bmat)
    return (A, S, R)


# speedup vs baseline: 24.9370x; 1.0559x over previous
import jax
import jax.numpy as jnp
from jax.experimental import pallas as pl
from jax.experimental.pallas import tpu as pltpu

_B, _N, _DIN, _RANK, _TOPK = 4, 2048, 256, 64, 32
_BLK = 256
_PREC = jax.lax.Precision.DEFAULT


def _r_body(x_ref, w_ref, b_ref, r_ref):
    x = x_ref[0]
    r = jax.lax.dot_general(x, w_ref[...], (((1,), (0,)), ((), ())),
                            preferred_element_type=jnp.float32,
                            precision=_PREC)
    r_ref[0] = r + b_ref[...]


def _s_body(rblk_ref, rfull_ref, bmat_ref, s_ref, a_ref):
    rblk = rblk_ref[0]
    rfull = rfull_ref[0]
    p = jax.lax.dot_general(rblk, bmat_ref[...], (((1,), (0,)), ((), ())),
                            preferred_element_type=jnp.float32,
                            precision=_PREC)
    s = jax.lax.dot_general(p, rfull, (((1,), (1,)), ((), ())),
                            preferred_element_type=jnp.float32,
                            precision=_PREC)
    s_ref[0] = s

    m = jnp.max(s, axis=-1, keepdims=True)
    e = jnp.exp(s - m)
    z = jnp.sum(e, axis=-1, keepdims=True)

    rows = pl.program_id(1) * _BLK + jax.lax.broadcasted_iota(
        jnp.int32, (_BLK, _N), 0)
    cols = jax.lax.broadcasted_iota(jnp.int32, (_BLK, _N), 1)
    nondiag = rows != cols
    neginf = jnp.float32(-jnp.inf)
    work = jnp.where(nondiag, s, neginf)

    sl = [work[:, j * 128:(j + 1) * 128] for j in range(16)]

    def _sort8_desc(v):
        net = [(0, 1), (2, 3), (4, 5), (6, 7),
               (0, 2), (1, 3), (4, 6), (5, 7),
               (1, 2), (5, 6), (0, 4), (3, 7),
               (1, 5), (2, 6), (1, 4), (3, 6),
               (2, 4), (3, 5), (3, 4)]
        v = list(v)
        for i, j in net:
            hi = jnp.maximum(v[i], v[j])
            lo = jnp.minimum(v[i], v[j])
            v[i], v[j] = hi, lo
        return v

    sa = _sort8_desc(sl[:8])
    sb = _sort8_desc(sl[8:])
    cand = [jnp.maximum(sa[i], sb[7 - i]) for i in range(8)]
    lane_min = cand[0]
    for j in range(1, 8):
        lane_min = jnp.minimum(lane_min, cand[j])
    tt = jnp.concatenate(cand, axis=-1)

    for _ in range(_TOPK - 1):
        v = jnp.max(tt, axis=-1, keepdims=True)
        tt = jnp.where(tt >= v, neginf, tt)
    t_fast = jnp.max(tt, axis=-1, keepdims=True)

    ok = jnp.logical_not(jnp.any(lane_min >= t_fast))

    def _slow(w):
        def body(_, ww):
            vv = jnp.max(ww, axis=-1, keepdims=True)
            return jnp.where(ww >= vv, neginf, ww)
        ww = jax.lax.fori_loop(0, _TOPK - 1, body, w)
        return jnp.max(ww, axis=-1, keepdims=True)

    t = jax.lax.cond(ok, lambda w: t_fast, _slow, work)

    keep = nondiag & (s >= t)
    ek = jnp.where(keep, e, jnp.float32(0.0))
    s1 = jnp.sum(ek, axis=-1, keepdims=True) / z
    c1 = jnp.maximum(s1, jnp.float32(1e-6))
    c2 = jnp.maximum(s1 / c1, jnp.float32(1e-6))
    scale = 1.0 / (z * c1 * c2)
    a_ref[0] = ek * scale


def kernel(X, W, b, Bmat):
    R = pl.pallas_call(
        _r_body,
        grid=(_B,),
        in_specs=[
            pl.BlockSpec((1, _N, _DIN), lambda i: (i, 0, 0)),
            pl.BlockSpec((_DIN, _RANK), lambda i: (0, 0)),
            pl.BlockSpec((1, _RANK), lambda i: (0, 0)),
        ],
        out_specs=pl.BlockSpec((1, _N, _RANK), lambda i: (i, 0, 0)),
        out_shape=jax.ShapeDtypeStruct((_B, _N, _RANK), jnp.float32),
    )(X, W, b.reshape(1, _RANK))

    S, A = pl.pallas_call(
        _s_body,
        grid=(_B, _N // _BLK),
        in_specs=[
            pl.BlockSpec((1, _BLK, _RANK), lambda i, j: (i, j, 0)),
            pl.BlockSpec((1, _N, _RANK), lambda i, j: (i, 0, 0)),
            pl.BlockSpec((_RANK, _RANK), lambda i, j: (0, 0)),
        ],
        out_specs=[
            pl.BlockSpec((1, _BLK, _N), lambda i, j: (i, j, 0)),
            pl.BlockSpec((1, _BLK, _N), lambda i, j: (i, j, 0)),
        ],
        out_shape=[
            jax.ShapeDtypeStruct((_B, _N, _N), jnp.float32),
            jax.ShapeDtypeStruct((_B, _N, _N), jnp.float32),
        ],
    )(R, R, Bmat)
    return (A, S, R)


# speedup vs baseline: 26.0972x; 1.0465x over previous
import jax
import jax.numpy as jnp
from jax.experimental import pallas as pl
from jax.experimental.pallas import tpu as pltpu

_B, _N, _DIN, _RANK, _TOPK = 4, 2048, 256, 64, 32
_BLK = 512
_PREC = jax.lax.Precision.DEFAULT


def _r_body(x_ref, w_ref, b_ref, r_ref):
    x = x_ref[0]
    r = jax.lax.dot_general(x, w_ref[...], (((1,), (0,)), ((), ())),
                            preferred_element_type=jnp.float32,
                            precision=_PREC)
    r_ref[0] = r + b_ref[...]


def _s_body(rblk_ref, rfull_ref, bmat_ref, s_ref, a_ref):
    rblk = rblk_ref[0]
    rfull = rfull_ref[0]
    p = jax.lax.dot_general(rblk, bmat_ref[...], (((1,), (0,)), ((), ())),
                            preferred_element_type=jnp.float32,
                            precision=_PREC)
    s = jax.lax.dot_general(p, rfull, (((1,), (1,)), ((), ())),
                            preferred_element_type=jnp.float32,
                            precision=_PREC)
    s_ref[0] = s

    m = jnp.max(s, axis=-1, keepdims=True)
    e = jnp.exp(s - m)
    z = jnp.sum(e, axis=-1, keepdims=True)

    rows = pl.program_id(1) * _BLK + jax.lax.broadcasted_iota(
        jnp.int32, (_BLK, _N), 0)
    cols = jax.lax.broadcasted_iota(jnp.int32, (_BLK, _N), 1)
    nondiag = rows != cols
    neginf = jnp.float32(-jnp.inf)
    work = jnp.where(nondiag, s, neginf)

    sl = [work[:, j * 128:(j + 1) * 128] for j in range(16)]

    def _sort8_desc(v):
        net = [(0, 1), (2, 3), (4, 5), (6, 7),
               (0, 2), (1, 3), (4, 6), (5, 7),
               (1, 2), (5, 6), (0, 4), (3, 7),
               (1, 5), (2, 6), (1, 4), (3, 6),
               (2, 4), (3, 5), (3, 4)]
        v = list(v)
        for i, j in net:
            hi = jnp.maximum(v[i], v[j])
            lo = jnp.minimum(v[i], v[j])
            v[i], v[j] = hi, lo
        return v

    sa = _sort8_desc(sl[:8])
    sb = _sort8_desc(sl[8:])
    cand = [jnp.maximum(sa[i], sb[7 - i]) for i in range(8)]
    lane_min = cand[0]
    for j in range(1, 8):
        lane_min = jnp.minimum(lane_min, cand[j])
    tt = jnp.concatenate(cand, axis=-1)

    for _ in range(_TOPK - 1):
        v = jnp.max(tt, axis=-1, keepdims=True)
        tt = jnp.where(tt >= v, neginf, tt)
    t_fast = jnp.max(tt, axis=-1, keepdims=True)

    ok = jnp.logical_not(jnp.any(lane_min >= t_fast))

    def _slow(w):
        def body(_, ww):
            vv = jnp.max(ww, axis=-1, keepdims=True)
            return jnp.where(ww >= vv, neginf, ww)
        ww = jax.lax.fori_loop(0, _TOPK - 1, body, w)
        return jnp.max(ww, axis=-1, keepdims=True)

    t = jax.lax.cond(ok, lambda w: t_fast, _slow, work)

    keep = nondiag & (s >= t)
    ek = jnp.where(keep, e, jnp.float32(0.0))
    s1 = jnp.sum(ek, axis=-1, keepdims=True) / z
    c1 = jnp.maximum(s1, jnp.float32(1e-6))
    c2 = jnp.maximum(s1 / c1, jnp.float32(1e-6))
    scale = 1.0 / (z * c1 * c2)
    a_ref[0] = ek * scale


def kernel(X, W, b, Bmat):
    R = pl.pallas_call(
        _r_body,
        grid=(_B,),
        in_specs=[
            pl.BlockSpec((1, _N, _DIN), lambda i: (i, 0, 0)),
            pl.BlockSpec((_DIN, _RANK), lambda i: (0, 0)),
            pl.BlockSpec((1, _RANK), lambda i: (0, 0)),
        ],
        out_specs=pl.BlockSpec((1, _N, _RANK), lambda i: (i, 0, 0)),
        out_shape=jax.ShapeDtypeStruct((_B, _N, _RANK), jnp.float32),
    )(X, W, b.reshape(1, _RANK))

    S, A = pl.pallas_call(
        _s_body,
        grid=(_B, _N // _BLK),
        in_specs=[
            pl.BlockSpec((1, _BLK, _RANK), lambda i, j: (i, j, 0)),
            pl.BlockSpec((1, _N, _RANK), lambda i, j: (i, 0, 0)),
            pl.BlockSpec((_RANK, _RANK), lambda i, j: (0, 0)),
        ],
        out_specs=[
            pl.BlockSpec((1, _BLK, _N), lambda i, j: (i, j, 0)),
            pl.BlockSpec((1, _BLK, _N), lambda i, j: (i, j, 0)),
        ],
        out_shape=[
            jax.ShapeDtypeStruct((_B, _N, _N), jnp.float32),
            jax.ShapeDtypeStruct((_B, _N, _N), jnp.float32),
        ],
    )(R, R, Bmat)
    return (A, S, R)
